# Initial kernel scaffold; baseline (speedup 1.0000x reference)
import jax
import jax.numpy as jnp
from jax.experimental import pallas as pl


def kernel(input, router_w, W_gate, W_up, W_down, lora_A, lora_B):
    raise NotImplementedError("write your pallas kernel here")


# speedup vs baseline: 1.8227x; 1.8227x over previous
import functools

import jax
import jax.numpy as jnp
from jax.experimental import pallas as pl
from jax.experimental.pallas import tpu as pltpu

_ALPHA = 32.0


def _fused_body(x_ref, rw_ref, wg_ref, wu_ref, wd_ref, la_ref, lb_ref,
                out_ref, comb_ref):
    e = pl.program_id(1)
    x = x_ref[...]

    @pl.when(e == 0)
    def _router_and_lora():
        logits = jnp.dot(x, rw_ref[...], preferred_element_type=jnp.float32)
        m = jnp.max(logits, axis=-1, keepdims=True)
        p = jnp.exp(logits - m)
        p = p / jnp.sum(p, axis=-1, keepdims=True)
        eidx = jax.lax.broadcasted_iota(jnp.int32, p.shape, 1)
        i1 = jnp.argmax(p, axis=-1)
        oh1 = eidx == i1[:, None]
        v1 = jnp.max(p, axis=-1)
        p2 = jnp.where(oh1, -jnp.inf, p)
        i2 = jnp.argmax(p2, axis=-1)
        oh2 = eidx == i2[:, None]
        v2 = jnp.max(p2, axis=-1)
        denom = v1 + v2
        comb = (jnp.where(oh1, (v1 / denom)[:, None], 0.0)
                + jnp.where(oh2, (v2 / denom)[:, None], 0.0))
        comb_ref[...] = comb.astype(jnp.float32)
        r = la_ref.shape[1]
        lora = jnp.dot(jnp.dot(x, la_ref[...], preferred_element_type=jnp.float32),
                       lb_ref[...], preferred_element_type=jnp.float32)
        out_ref[...] = lora * (_ALPHA / r)

    g = jnp.dot(x, wg_ref[0], preferred_element_type=jnp.float32)
    u = jnp.dot(x, wu_ref[0], preferred_element_type=jnp.float32)
    hdn = (g * jax.nn.sigmoid(g)) * u
    y = jnp.dot(hdn, wd_ref[0], preferred_element_type=jnp.float32)
    comb = comb_ref[...]
    lane = jax.lax.broadcasted_iota(jnp.int32, comb.shape, 1)
    ce = jnp.sum(jnp.where(lane == e, comb, 0.0), axis=1, keepdims=True)
    out_ref[...] += y * ce


def kernel(input, router_w, W_gate, W_up, W_down, lora_A, lora_B):
    b, s, h = input.shape
    t = b * s
    e = router_w.shape[1]
    esz = W_gate.shape[2]
    r = lora_A.shape[1]
    x = input.reshape(t, h)

    tb = min(512, t)
    grid = (t // tb, e)

    out = pl.pallas_call(
        _fused_body,
        grid=grid,
        in_specs=[
            pl.BlockSpec((tb, h), lambda i, j: (i, 0)),
            pl.BlockSpec((h, e), lambda i, j: (0, 0)),
            pl.BlockSpec((1, h, esz), lambda i, j: (j, 0, 0)),
            pl.BlockSpec((1, h, esz), lambda i, j: (j, 0, 0)),
            pl.BlockSpec((1, esz, h), lambda i, j: (j, 0, 0)),
            pl.BlockSpec((h, r), lambda i, j: (0, 0)),
            pl.BlockSpec((r, h), lambda i, j: (0, 0)),
        ],
        out_specs=pl.BlockSpec((tb, h), lambda i, j: (i, 0)),
        out_shape=jax.ShapeDtypeStruct((t, h), jnp.float32),
        scratch_shapes=[pltpu.VMEM((tb, e), jnp.float32)],
        compiler_params=pltpu.CompilerParams(
            dimension_semantics=("arbitrary", "arbitrary"),
        ),
    )(x, router_w, W_gate, W_up, W_down, lora_A, lora_B)
    return out.reshape(b, s, h)
